# baseline (speedup 1.0000x reference)
import functools

import jax
import jax.numpy as jnp
from jax import lax
from jax.experimental import pallas as pl
from jax.experimental.pallas import tpu as pltpu
from jax.experimental.pallas import tpu_sc as plsc

N = 10000
NP = 10240
E = 160000
R = 8
D = 256
HALF = D // 2
LAYERS = 3
B = 256
NEG = 32

NC, NS, LANES = 2, 16, 16
NW = NC * NS
CHUNK = 128
NCH = 40
E_WP = NCH * CHUNK
EP = NW * E_WP
KREP = 512
ROWS_SUB = NP // NS
ZCH = 64
NZ = ROWS_SUB // ZCH

BN = 400
NB = N // BN

TRI = B * NEG
T_W = TRI // NW
KS = 64
NKS = T_W // KS


@functools.lru_cache(maxsize=None)
def _sc_mesh():
    return plsc.VectorSubcoreMesh(core_axis_name="c", subcore_axis_name="s",
                                  num_cores=NC, num_subcores=NS)


@functools.lru_cache(maxsize=None)
def _scatter_kernel(tab_rows):
    del tab_rows
    return pl.kernel(
        _scatter_body,
        out_type=tuple(jax.ShapeDtypeStruct((NP, HALF), jnp.float32)
                       for _ in range(NC)),
        mesh=_sc_mesh(),
        scratch_types=[
            pltpu.VMEM((CHUNK,), jnp.int32),
            pltpu.VMEM((CHUNK,), jnp.int32),
            pltpu.VMEM((CHUNK,), jnp.int32),
            pltpu.VMEM((CHUNK,), jnp.int32),
            pltpu.VMEM((CHUNK, HALF), jnp.float32),
            pltpu.VMEM((CHUNK, HALF), jnp.float32),
            pltpu.VMEM((ZCH, HALF), jnp.float32),
            pltpu.VMEM_SHARED((NP, HALF), jnp.float32),
            pltpu.SemaphoreType.DMA,
            pltpu.SemaphoreType.DMA,
        ],
    )


def _scatter_body(tab, idx_hbm, dst_hbm, zrow, out0, out1,
                  i0, i1, d0, d1, r0, r1, zbuf, agg_sh, sem0, sem1):
    c = lax.axis_index("c")
    s = lax.axis_index("s")
    base = s * ROWS_SUB
    eoff0 = (c * NS + s) * E_WP

    pltpu.sync_copy(zrow, zbuf)
    for j in range(NZ):
        pltpu.sync_copy(zbuf, agg_sh.at[pl.ds(base + j * ZCH, ZCH)])
    plsc.subcore_barrier()

    pltpu.sync_copy(idx_hbm.at[pl.ds(eoff0, CHUNK)], i0)
    pltpu.sync_copy(dst_hbm.at[pl.ds(eoff0, CHUNK)], d0)
    pltpu.async_copy(tab.at[i0], r0, sem0)

    @pl.loop(0, NCH, step=2)
    def _pair(j):
        pltpu.sync_copy(idx_hbm.at[pl.ds(eoff0 + (j + 1) * CHUNK, CHUNK)], i1)
        pltpu.sync_copy(dst_hbm.at[pl.ds(eoff0 + (j + 1) * CHUNK, CHUNK)], d1)
        pltpu.async_copy(tab.at[i1], r1, sem1)
        pltpu.make_async_copy(tab.at[i0], r0, sem0).wait()
        pltpu.sync_copy(r0, agg_sh.at[d0], add=True)

        @pl.when(j + 2 < NCH)
        def _():
            pltpu.sync_copy(idx_hbm.at[pl.ds(eoff0 + (j + 2) * CHUNK, CHUNK)], i0)
            pltpu.sync_copy(dst_hbm.at[pl.ds(eoff0 + (j + 2) * CHUNK, CHUNK)], d0)
            pltpu.async_copy(tab.at[i0], r0, sem0)

        pltpu.make_async_copy(tab.at[i1], r1, sem1).wait()
        pltpu.sync_copy(r1, agg_sh.at[d1], add=True)

    plsc.subcore_barrier()

    @pl.when(c == 0)
    def _():
        for j in range(NZ):
            pltpu.sync_copy(agg_sh.at[pl.ds(base + j * ZCH, ZCH)], zbuf)
            pltpu.sync_copy(zbuf, out0.at[pl.ds(base + j * ZCH, ZCH)])

    @pl.when(c == 1)
    def _():
        for j in range(NZ):
            pltpu.sync_copy(agg_sh.at[pl.ds(base + j * ZCH, ZCH)], zbuf)
            pltpu.sync_copy(zbuf, out1.at[pl.ds(base + j * ZCH, ZCH)])


def _scatter_pass(tab, idx, dstp, zrow):
    return _scatter_kernel(tab.shape[0])(tab, idx, dstp, zrow)


def _xw_half_body(x_ref, w_ref, o_ref):
    o_ref[...] = jnp.dot(x_ref[...], w_ref[0],
                         preferred_element_type=jnp.float32)


def _xw_half(x, w_half):
    return pl.pallas_call(
        _xw_half_body,
        grid=(NB, R),
        in_specs=[
            pl.BlockSpec((BN, D), lambda i, r: (i, 0)),
            pl.BlockSpec((1, D, HALF), lambda i, r: (r, 0, 0)),
        ],
        out_specs=pl.BlockSpec((BN, HALF), lambda i, r: (r * NB + i, 0)),
        out_shape=jax.ShapeDtypeStruct((R * N, HALF), jnp.float32),
    )(x, w_half)


def _combine0_body(c0_ref, c1_ref, wrel_ref, ws_ref, b_ref, sc_ref, bi_ref,
                   out_ref):
    cnt = c0_ref[...] + c1_ref[...]
    deg = jnp.maximum(jnp.sum(cnt, axis=-1, keepdims=True), 1.0)
    cs = jnp.sum(wrel_ref[...], axis=1)
    cs128 = jnp.concatenate(
        [cs, jnp.zeros((HALF - R, D), jnp.float32)], axis=0)
    agg = jnp.dot(cnt, cs128, preferred_element_type=jnp.float32) / deg
    selfrow = jnp.sum(ws_ref[...], axis=0, keepdims=True)
    h = agg + selfrow + b_ref[...]
    mu = jnp.mean(h, axis=-1, keepdims=True)
    hc = h - mu
    var = jnp.mean(hc * hc, axis=-1, keepdims=True)
    h = hc * lax.rsqrt(var + 1e-5) * sc_ref[...] + bi_ref[...]
    out_ref[...] = jnp.maximum(h, 0.0) + 1.0


def _combine0(c0, c1, wrel, ws, bv, scv, biv):
    cspec = pl.BlockSpec((BN, HALF), lambda i: (i, 0))
    return pl.pallas_call(
        _combine0_body,
        grid=(NB,),
        in_specs=[
            cspec, cspec,
            pl.BlockSpec((R, D, D), lambda i: (0, 0, 0)),
            pl.BlockSpec((D, D), lambda i: (0, 0)),
            pl.BlockSpec((1, D), lambda i: (0, 0)),
            pl.BlockSpec((1, D), lambda i: (0, 0)),
            pl.BlockSpec((1, D), lambda i: (0, 0)),
        ],
        out_specs=pl.BlockSpec((BN, D), lambda i: (i, 0)),
        out_shape=jax.ShapeDtypeStruct((N, D), jnp.float32),
    )(c0, c1, wrel, ws, bv, scv, biv)


def _combine_body(a00_ref, a01_ref, a10_ref, a11_ref, c0_ref, c1_ref,
                  x_ref, ws_ref, b_ref, sc_ref, bi_ref, out_ref):
    cnt = c0_ref[...] + c1_ref[...]
    deg = jnp.maximum(jnp.sum(cnt, axis=-1, keepdims=True), 1.0)
    agg = jnp.concatenate([a00_ref[...] + a01_ref[...],
                           a10_ref[...] + a11_ref[...]], axis=-1) / deg
    x = x_ref[...]
    h = agg + jnp.dot(x, ws_ref[...], preferred_element_type=jnp.float32) + b_ref[...]
    mu = jnp.mean(h, axis=-1, keepdims=True)
    hc = h - mu
    var = jnp.mean(hc * hc, axis=-1, keepdims=True)
    h = hc * lax.rsqrt(var + 1e-5) * sc_ref[...] + bi_ref[...]
    out_ref[...] = jnp.maximum(h, 0.0) + x


def _combine(a00, a01, a10, a11, c0, c1, x, ws, bv, scv, biv):
    hspec = pl.BlockSpec((BN, HALF), lambda i: (i, 0))
    return pl.pallas_call(
        _combine_body,
        grid=(NB,),
        in_specs=[
            hspec, hspec, hspec, hspec, hspec, hspec,
            pl.BlockSpec((BN, D), lambda i: (i, 0)),
            pl.BlockSpec((D, D), lambda i: (0, 0)),
            pl.BlockSpec((1, D), lambda i: (0, 0)),
            pl.BlockSpec((1, D), lambda i: (0, 0)),
            pl.BlockSpec((1, D), lambda i: (0, 0)),
        ],
        out_specs=pl.BlockSpec((BN, D), lambda i: (i, 0)),
        out_shape=jax.ShapeDtypeStruct((N, D), jnp.float32),
    )(a00, a01, a10, a11, c0, c1, x, ws, bv, scv, biv)


@functools.lru_cache(maxsize=None)
def _score_kernel():
    return pl.kernel(
        _score_body,
        out_type=jax.ShapeDtypeStruct((TRI, LANES), jnp.float32),
        mesh=_sc_mesh(),
        scratch_types=[
            pltpu.VMEM((KS,), jnp.int32),
            pltpu.VMEM((KS,), jnp.int32),
            pltpu.VMEM((KS,), jnp.int32),
            pltpu.VMEM((KS, D), jnp.float32),
            pltpu.VMEM((KS, D), jnp.float32),
            pltpu.VMEM((KS, D), jnp.float32),
            pltpu.VMEM((T_W, LANES), jnp.float32),
            pltpu.SemaphoreType.DMA,
        ],
    )


def _score_body(x_hbm, rel_hbm, h_hbm, t_hbm, r_hbm, out,
                hi, ti, ri, hrow, trow, rrow, outv, sem):
    c = lax.axis_index("c")
    s = lax.axis_index("s")
    wid = s * NC + c

    @pl.loop(0, NKS)
    def _chunk(j):
        toff = wid * T_W + j * KS
        pltpu.sync_copy(h_hbm.at[pl.ds(toff, KS)], hi)
        pltpu.sync_copy(t_hbm.at[pl.ds(toff, KS)], ti)
        pltpu.sync_copy(r_hbm.at[pl.ds(toff, KS)], ri)
        pltpu.async_copy(x_hbm.at[hi], hrow, sem).wait()
        pltpu.async_copy(x_hbm.at[ti], trow, sem).wait()
        pltpu.async_copy(rel_hbm.at[ri], rrow, sem).wait()

        @pl.loop(0, KS)
        def _tri(k):
            acc = hrow[k, pl.ds(0, LANES)] * rrow[k, pl.ds(0, LANES)] \
                * trow[k, pl.ds(0, LANES)]
            for t in range(1, D // LANES):
                o = t * LANES
                acc = acc + hrow[k, pl.ds(o, LANES)] * rrow[k, pl.ds(o, LANES)] \
                    * trow[k, pl.ds(o, LANES)]
            outv[j * KS + k] = acc

    pltpu.sync_copy(outv, out.at[pl.ds(wid * T_W, T_W)])


def _score_reduce_body(p_ref, out_ref):
    s = jnp.sum(p_ref[...], axis=-1)
    out_ref[...] = s.reshape(TRI // 128, 128)


def _score_reduce(partials):
    return pl.pallas_call(
        _score_reduce_body,
        in_specs=[pl.BlockSpec((TRI, LANES), lambda: (0, 0))],
        out_specs=pl.BlockSpec((TRI // 128, 128), lambda: (0, 0)),
        out_shape=jax.ShapeDtypeStruct((TRI // 128, 128), jnp.float32),
    )(partials)


def kernel(W_rel, W_self, b, ln_scale, ln_bias, rel_emb, edge_index, edge_type, batch):
    src = edge_index[0].astype(jnp.int32)
    dst = edge_index[1].astype(jnp.int32)
    et = edge_type.astype(jnp.int32)
    pad = EP - E
    gidx = jnp.concatenate([et * N + src, jnp.zeros((pad,), jnp.int32)])
    cidx = jnp.concatenate([et + R * (jnp.arange(E, dtype=jnp.int32) % KREP),
                            jnp.zeros((pad,), jnp.int32)])
    dstp = jnp.concatenate([dst, jnp.full((pad,), N, jnp.int32)])
    zrow = jnp.zeros((ZCH, HALF), jnp.float32)
    onehot = jnp.tile(jnp.eye(R, HALF, dtype=jnp.float32), (KREP, 1))

    cnt0, cnt1 = _scatter_pass(onehot, cidx, dstp, zrow)
    x = _combine0(cnt0[:N], cnt1[:N], W_rel[0], W_self[0], b[0][None],
                  ln_scale[0][None], ln_bias[0][None])
    for l in range(1, LAYERS):
        h0 = _xw_half(x, W_rel[l][:, :, :HALF])
        a00, a01 = _scatter_pass(h0, gidx, dstp, zrow)
        h1 = _xw_half(x, W_rel[l][:, :, HALF:])
        a10, a11 = _scatter_pass(h1, gidx, dstp, zrow)
        x = _combine(a00[:N], a01[:N], a10[:N], a11[:N], cnt0[:N], cnt1[:N],
                     x, W_self[l], b[l][None], ln_scale[l][None], ln_bias[l][None])

    hh = batch[:, :, 0].reshape(TRI).astype(jnp.int32)
    tt = batch[:, :, 1].reshape(TRI).astype(jnp.int32)
    rr = batch[:, :, 2].reshape(TRI).astype(jnp.int32)
    partials = _score_kernel()(x, rel_emb, hh, tt, rr)
    return _score_reduce(partials).reshape(B, NEG)


# speedup vs baseline: 1.3288x; 1.3288x over previous
import functools

import jax
import jax.numpy as jnp
from jax import lax
from jax.experimental import pallas as pl
from jax.experimental.pallas import tpu as pltpu
from jax.experimental.pallas import tpu_sc as plsc

N = 10000
NP = 10240
E = 160000
R = 8
D = 256
HALF = D // 2
LAYERS = 3
B = 256
NEG = 32

NC, NS, LANES = 2, 16, 16
NW = NC * NS
CHUNK = 128
NCH = 80
E_SUBP = NCH * CHUNK
NCH_C = NCH // 2
E_WP = NCH_C * CHUNK
EP = NS * E_SUBP
KREP = 512
ROWS_SUB = NP // NS
ZCH = 64
NZ = ROWS_SUB // ZCH

BN = 400
NB = N // BN

TRI = B * NEG
T_W = TRI // NW
KS = 64
NKS = T_W // KS


@functools.lru_cache(maxsize=None)
def _sc_mesh():
    return plsc.VectorSubcoreMesh(core_axis_name="c", subcore_axis_name="s",
                                  num_cores=NC, num_subcores=NS)


_SC_SCRATCH = [
    pltpu.VMEM((CHUNK,), jnp.int32),
    pltpu.VMEM((CHUNK,), jnp.int32),
    pltpu.VMEM((CHUNK,), jnp.int32),
    pltpu.VMEM((CHUNK,), jnp.int32),
    pltpu.VMEM((CHUNK, HALF), jnp.float32),
    pltpu.VMEM((CHUNK, HALF), jnp.float32),
    pltpu.VMEM((ZCH, HALF), jnp.float32),
    pltpu.VMEM_SHARED((NP, HALF), jnp.float32),
    pltpu.SemaphoreType.DMA,
    pltpu.SemaphoreType.DMA,
]


def _zero_spmem(zrow, zbuf, sh, base):
    pltpu.sync_copy(zrow, zbuf)
    for j in range(NZ):
        pltpu.sync_copy(zbuf, sh.at[pl.ds(base + j * ZCH, ZCH)])


def _flush_spmem(sh, zbuf, out, base):
    for j in range(NZ):
        pltpu.sync_copy(sh.at[pl.ds(base + j * ZCH, ZCH)], zbuf)
        pltpu.sync_copy(zbuf, out.at[pl.ds(base + j * ZCH, ZCH)])


def _gs_chunks(tab, idx_hbm, dst_hbm, eoff0,
               i0, i1, d0, d1, r0, r1, agg_sh, sem0, sem1, nch):
    pltpu.sync_copy(idx_hbm.at[pl.ds(eoff0, CHUNK)], i0)
    pltpu.sync_copy(dst_hbm.at[pl.ds(eoff0, CHUNK)], d0)
    pltpu.async_copy(tab.at[i0], r0, sem0)

    @pl.loop(0, nch, step=2)
    def _pair(j):
        pltpu.sync_copy(idx_hbm.at[pl.ds(eoff0 + (j + 1) * CHUNK, CHUNK)], i1)
        pltpu.sync_copy(dst_hbm.at[pl.ds(eoff0 + (j + 1) * CHUNK, CHUNK)], d1)
        pltpu.async_copy(tab.at[i1], r1, sem1)
        pltpu.make_async_copy(tab.at[i0], r0, sem0).wait()
        pltpu.sync_copy(r0, agg_sh.at[d0], add=True)

        @pl.when(j + 2 < nch)
        def _():
            pltpu.sync_copy(idx_hbm.at[pl.ds(eoff0 + (j + 2) * CHUNK, CHUNK)], i0)
            pltpu.sync_copy(dst_hbm.at[pl.ds(eoff0 + (j + 2) * CHUNK, CHUNK)], d0)
            pltpu.async_copy(tab.at[i0], r0, sem0)

        pltpu.make_async_copy(tab.at[i1], r1, sem1).wait()
        pltpu.sync_copy(r1, agg_sh.at[d1], add=True)


@functools.lru_cache(maxsize=None)
def _counts_kernel():
    return pl.kernel(
        _counts_body,
        out_type=tuple(jax.ShapeDtypeStruct((NP, HALF), jnp.float32)
                       for _ in range(NC)),
        mesh=_sc_mesh(),
        scratch_types=list(_SC_SCRATCH),
    )


def _counts_body(onehot, cidx, dstx, zrow, c0_out, c1_out,
                 i0, i1, d0, d1, r0, r1, zbuf, cnt_sh, sem0, sem1):
    c = lax.axis_index("c")
    s = lax.axis_index("s")
    base = s * ROWS_SUB

    _zero_spmem(zrow, zbuf, cnt_sh, base)
    plsc.subcore_barrier()
    _gs_chunks(onehot, cidx, dstx, (c * NS + s) * E_WP,
               i0, i1, d0, d1, r0, r1, cnt_sh, sem0, sem1, NCH_C)
    plsc.subcore_barrier()

    @pl.when(c == 0)
    def _():
        _flush_spmem(cnt_sh, zbuf, c0_out, base)

    @pl.when(c == 1)
    def _():
        _flush_spmem(cnt_sh, zbuf, c1_out, base)


@functools.lru_cache(maxsize=None)
def _edge_pass_kernel():
    return pl.kernel(
        _edge_pass_body,
        out_type=tuple(jax.ShapeDtypeStruct((NP, HALF), jnp.float32)
                       for _ in range(NC)),
        mesh=_sc_mesh(),
        scratch_types=list(_SC_SCRATCH),
    )


def _edge_pass_body(th0, th1, gidx, dstx, zrow,
                    a0, a1,
                    i0, i1, d0, d1, r0, r1, zbuf, agg_sh, sem0, sem1):
    c = lax.axis_index("c")
    s = lax.axis_index("s")
    base = s * ROWS_SUB

    _zero_spmem(zrow, zbuf, agg_sh, base)
    plsc.subcore_barrier()

    @pl.when(c == 0)
    def _():
        _gs_chunks(th0, gidx, dstx, s * E_SUBP,
                   i0, i1, d0, d1, r0, r1, agg_sh, sem0, sem1, NCH)

    @pl.when(c == 1)
    def _():
        _gs_chunks(th1, gidx, dstx, s * E_SUBP,
                   i0, i1, d0, d1, r0, r1, agg_sh, sem0, sem1, NCH)

    plsc.subcore_barrier()

    @pl.when(c == 0)
    def _():
        _flush_spmem(agg_sh, zbuf, a0, base)

    @pl.when(c == 1)
    def _():
        _flush_spmem(agg_sh, zbuf, a1, base)


def _xw_body(x_ref, w_ref, o0, o1):
    y = jnp.dot(x_ref[...], w_ref[0], preferred_element_type=jnp.float32)
    o0[...] = y[:, :HALF]
    o1[...] = y[:, HALF:]


def _xw(x, w):
    hspec = pl.BlockSpec((BN, HALF), lambda i, r: (r * NB + i, 0))
    hshape = jax.ShapeDtypeStruct((R * N, HALF), jnp.float32)
    return pl.pallas_call(
        _xw_body,
        grid=(NB, R),
        in_specs=[
            pl.BlockSpec((BN, D), lambda i, r: (i, 0)),
            pl.BlockSpec((1, D, D), lambda i, r: (r, 0, 0)),
        ],
        out_specs=[hspec, hspec],
        out_shape=[hshape, hshape],
    )(x, w)


def _combine0_body(c0_ref, c1_ref, wrel_ref, ws_ref, b_ref, sc_ref, bi_ref,
                   out_ref):
    cnt = c0_ref[...] + c1_ref[...]
    deg = jnp.maximum(jnp.sum(cnt, axis=-1, keepdims=True), 1.0)
    cs = jnp.sum(wrel_ref[...], axis=1)
    cs128 = jnp.concatenate(
        [cs, jnp.zeros((HALF - R, D), jnp.float32)], axis=0)
    agg = jnp.dot(cnt, cs128, preferred_element_type=jnp.float32) / deg
    selfrow = jnp.sum(ws_ref[...], axis=0, keepdims=True)
    h = agg + selfrow + b_ref[...]
    mu = jnp.mean(h, axis=-1, keepdims=True)
    hc = h - mu
    var = jnp.mean(hc * hc, axis=-1, keepdims=True)
    h = hc * lax.rsqrt(var + 1e-5) * sc_ref[...] + bi_ref[...]
    out_ref[...] = jnp.maximum(h, 0.0) + 1.0


def _combine0(c0, c1, wrel, ws, bv, scv, biv):
    cspec = pl.BlockSpec((BN, HALF), lambda i: (i, 0))
    return pl.pallas_call(
        _combine0_body,
        grid=(NB,),
        in_specs=[
            cspec, cspec,
            pl.BlockSpec((R, D, D), lambda i: (0, 0, 0)),
            pl.BlockSpec((D, D), lambda i: (0, 0)),
            pl.BlockSpec((1, D), lambda i: (0, 0)),
            pl.BlockSpec((1, D), lambda i: (0, 0)),
            pl.BlockSpec((1, D), lambda i: (0, 0)),
        ],
        out_specs=pl.BlockSpec((BN, D), lambda i: (i, 0)),
        out_shape=jax.ShapeDtypeStruct((N, D), jnp.float32),
    )(c0, c1, wrel, ws, bv, scv, biv)


def _combine_body(a0_ref, a1_ref, c0_ref, c1_ref, x_ref, ws_ref,
                  b_ref, sc_ref, bi_ref, out_ref):
    cnt = c0_ref[...] + c1_ref[...]
    deg = jnp.maximum(jnp.sum(cnt, axis=-1, keepdims=True), 1.0)
    agg = jnp.concatenate([a0_ref[...], a1_ref[...]], axis=-1) / deg
    x = x_ref[...]
    h = agg + jnp.dot(x, ws_ref[...], preferred_element_type=jnp.float32) + b_ref[...]
    mu = jnp.mean(h, axis=-1, keepdims=True)
    hc = h - mu
    var = jnp.mean(hc * hc, axis=-1, keepdims=True)
    h = hc * lax.rsqrt(var + 1e-5) * sc_ref[...] + bi_ref[...]
    out_ref[...] = jnp.maximum(h, 0.0) + x


def _combine(a0, a1, c0, c1, x, ws, bv, scv, biv):
    hspec = pl.BlockSpec((BN, HALF), lambda i: (i, 0))
    return pl.pallas_call(
        _combine_body,
        grid=(NB,),
        in_specs=[
            hspec, hspec, hspec, hspec,
            pl.BlockSpec((BN, D), lambda i: (i, 0)),
            pl.BlockSpec((D, D), lambda i: (0, 0)),
            pl.BlockSpec((1, D), lambda i: (0, 0)),
            pl.BlockSpec((1, D), lambda i: (0, 0)),
            pl.BlockSpec((1, D), lambda i: (0, 0)),
        ],
        out_specs=pl.BlockSpec((BN, D), lambda i: (i, 0)),
        out_shape=jax.ShapeDtypeStruct((N, D), jnp.float32),
    )(a0, a1, c0, c1, x, ws, bv, scv, biv)


@functools.lru_cache(maxsize=None)
def _score_kernel():
    return pl.kernel(
        _score_body,
        out_type=jax.ShapeDtypeStruct((TRI, LANES), jnp.float32),
        mesh=_sc_mesh(),
        scratch_types=[
            pltpu.VMEM((KS,), jnp.int32),
            pltpu.VMEM((KS,), jnp.int32),
            pltpu.VMEM((KS,), jnp.int32),
            pltpu.VMEM((KS, D), jnp.float32),
            pltpu.VMEM((KS, D), jnp.float32),
            pltpu.VMEM((KS, D), jnp.float32),
            pltpu.VMEM((T_W, LANES), jnp.float32),
            pltpu.SemaphoreType.DMA,
        ],
    )


def _score_body(x_hbm, rel_hbm, h_hbm, t_hbm, r_hbm, out,
                hi, ti, ri, hrow, trow, rrow, outv, sem):
    c = lax.axis_index("c")
    s = lax.axis_index("s")
    wid = s * NC + c

    @pl.loop(0, NKS)
    def _chunk(j):
        toff = wid * T_W + j * KS
        pltpu.sync_copy(h_hbm.at[pl.ds(toff, KS)], hi)
        pltpu.sync_copy(t_hbm.at[pl.ds(toff, KS)], ti)
        pltpu.sync_copy(r_hbm.at[pl.ds(toff, KS)], ri)
        pltpu.async_copy(x_hbm.at[hi], hrow, sem).wait()
        pltpu.async_copy(x_hbm.at[ti], trow, sem).wait()
        pltpu.async_copy(rel_hbm.at[ri], rrow, sem).wait()

        @pl.loop(0, KS)
        def _tri(k):
            acc = hrow[k, pl.ds(0, LANES)] * rrow[k, pl.ds(0, LANES)] \
                * trow[k, pl.ds(0, LANES)]
            for t in range(1, D // LANES):
                o = t * LANES
                acc = acc + hrow[k, pl.ds(o, LANES)] * rrow[k, pl.ds(o, LANES)] \
                    * trow[k, pl.ds(o, LANES)]
            outv[j * KS + k] = acc

    pltpu.sync_copy(outv, out.at[pl.ds(wid * T_W, T_W)])


def _score_reduce_body(p_ref, out_ref):
    s = jnp.sum(p_ref[...], axis=-1)
    out_ref[...] = s.reshape(TRI // 128, 128)


def _score_reduce(partials):
    return pl.pallas_call(
        _score_reduce_body,
        in_specs=[pl.BlockSpec((TRI, LANES), lambda: (0, 0))],
        out_specs=pl.BlockSpec((TRI // 128, 128), lambda: (0, 0)),
        out_shape=jax.ShapeDtypeStruct((TRI // 128, 128), jnp.float32),
    )(partials)


def kernel(W_rel, W_self, b, ln_scale, ln_bias, rel_emb, edge_index, edge_type, batch):
    src = edge_index[0].astype(jnp.int32)
    dst = edge_index[1].astype(jnp.int32)
    et = edge_type.astype(jnp.int32)
    pad = EP - E
    gidx = jnp.concatenate([et * N + src, jnp.zeros((pad,), jnp.int32)])
    cidx = jnp.concatenate([et + R * (jnp.arange(E, dtype=jnp.int32) % KREP),
                            jnp.zeros((pad,), jnp.int32)])
    dstp = jnp.concatenate([dst, jnp.full((pad,), N, jnp.int32)])
    zrow = jnp.zeros((ZCH, HALF), jnp.float32)
    onehot = jnp.tile(jnp.eye(R, HALF, dtype=jnp.float32), (KREP, 1))

    cnt0, cnt1 = _counts_kernel()(onehot, cidx, dstp, zrow)
    x = _combine0(cnt0[:N], cnt1[:N], W_rel[0], W_self[0], b[0][None],
                  ln_scale[0][None], ln_bias[0][None])
    for l in range(1, LAYERS):
        h0, h1 = _xw(x, W_rel[l])
        a0, a1 = _edge_pass_kernel()(h0, h1, gidx, dstp, zrow)
        x = _combine(a0[:N], a1[:N], cnt0[:N], cnt1[:N], x, W_self[l],
                     b[l][None], ln_scale[l][None], ln_bias[l][None])

    hh = batch[:, :, 0].reshape(TRI).astype(jnp.int32)
    tt = batch[:, :, 1].reshape(TRI).astype(jnp.int32)
    rr = batch[:, :, 2].reshape(TRI).astype(jnp.int32)
    partials = _score_kernel()(x, rel_emb, hh, tt, rr)
    return _score_reduce(partials).reshape(B, NEG)
